# baseline (speedup 1.0000x reference)
import functools

import jax
import jax.numpy as jnp
from jax import lax
from jax.experimental import pallas as pl
from jax.experimental.pallas import tpu as pltpu
from jax.experimental.pallas import tpu_sc as plsc

_BATCH = 256
_N_TOK = 77
_PAD_TOK = 80
_D = 768
_PROWS = _BATCH * _PAD_TOK
_CH = 32
_LANES = 16


def _sc_embed(tokens_pflat, table, pos_pad):
    info = plsc.get_sparse_core_info()
    num_workers = info.num_cores * info.num_subcores
    rows_per_w = _PROWS // num_workers
    num_chunks = rows_per_w // _CH
    num_pairs = num_chunks // 2

    mesh = plsc.VectorSubcoreMesh(core_axis_name="c", subcore_axis_name="s")

    @functools.partial(
        pl.kernel,
        mesh=mesh,
        out_type=jax.ShapeDtypeStruct((_PROWS, _D), jnp.float32),
        scratch_types=[
            pltpu.VMEM((rows_per_w,), jnp.int32),
            pltpu.VMEM((_CH, _D), jnp.float32),
            pltpu.VMEM((_CH, _D), jnp.float32),
            pltpu.VMEM((_PAD_TOK, _D), jnp.float32),
            pltpu.SemaphoreType.DMA,
            pltpu.SemaphoreType.DMA,
            pltpu.SemaphoreType.DMA,
            pltpu.SemaphoreType.DMA,
        ],
    )
    def body(
        tok_hbm, tab_hbm, pos_hbm, out_hbm,
        idx_v, buf0, buf1, pos_v, gsem0, gsem1, wsem0, wsem1,
    ):
        wid = lax.axis_index("s") * info.num_cores + lax.axis_index("c")
        r0 = wid * rows_per_w
        pltpu.sync_copy(tok_hbm.at[pl.ds(r0, rows_per_w)], idx_v)
        pltpu.sync_copy(pos_hbm, pos_v)

        def start_gather(c, buf, sem):
            pltpu.async_copy(
                tab_hbm.at[idx_v.at[pl.ds(c * _CH, _CH)]], buf, sem
            )

        def wait_gather(buf, sem):
            pltpu.make_async_copy(
                tab_hbm.at[idx_v.at[pl.ds(0, _CH)]], buf, sem
            ).wait()

        def start_wb(c, buf, sem):
            pltpu.async_copy(buf, out_hbm.at[pl.ds(r0 + c * _CH, _CH)], sem)

        def wait_wb(c, buf, sem):
            pltpu.make_async_copy(
                buf, out_hbm.at[pl.ds(r0 + c * _CH, _CH)], sem
            ).wait()

        def add_pos(c, buf):
            p0 = lax.rem(r0 + c * _CH, _PAD_TOK)

            @plsc.parallel_loop(0, _CH)
            def _(i):
                p = lax.rem(p0 + i, _PAD_TOK)

                @plsc.parallel_loop(0, _D // _LANES, unroll=8)
                def _(k):
                    sl = pl.ds(k * _LANES, _LANES)
                    buf[i, sl] = buf[i, sl] + pos_v[p, sl]

        start_gather(0, buf0, gsem0)

        def pair(t, carry):
            e = 2 * t
            o = e + 1

            @pl.when(t > 0)
            def _():
                wait_wb(o - 2, buf1, wsem1)

            start_gather(o, buf1, gsem1)
            wait_gather(buf0, gsem0)
            add_pos(e, buf0)
            start_wb(e, buf0, wsem0)
            wait_gather(buf1, gsem1)
            add_pos(o, buf1)
            start_wb(o, buf1, wsem1)
            wait_wb(e, buf0, wsem0)

            @pl.when(t < num_pairs - 1)
            def _():
                start_gather(e + 2, buf0, gsem0)

            return carry

        lax.fori_loop(0, num_pairs, pair, 0)
        wait_wb(num_chunks - 1, buf1, wsem1)

    return body(tokens_pflat, table, pos_pad)


def kernel(tokens, token_embedding, positional_embedding):
    tokens_pad = jnp.pad(
        tokens.astype(jnp.int32), ((0, 0), (0, _PAD_TOK - _N_TOK))
    ).reshape(-1)
    pos_pad = jnp.pad(
        positional_embedding, ((0, _PAD_TOK - _N_TOK), (0, 0))
    )
    out = _sc_embed(tokens_pad, token_embedding, pos_pad)
    return out.reshape(_BATCH, _PAD_TOK, _D)[:, :_N_TOK, :]


# speedup vs baseline: 1.0725x; 1.0725x over previous
import functools

import jax
import jax.numpy as jnp
from jax import lax
from jax.experimental import pallas as pl
from jax.experimental.pallas import tpu as pltpu
from jax.experimental.pallas import tpu_sc as plsc

_BATCH = 256
_N_TOK = 77
_PAD_TOK = 80
_D = 768
_LANES = 16


def _sc_embed(tokens_pflat, table, pos_pad, rowmap):
    info = plsc.get_sparse_core_info()
    num_workers = info.num_cores * info.num_subcores
    b_per_w = _BATCH // num_workers

    mesh = plsc.VectorSubcoreMesh(core_axis_name="c", subcore_axis_name="s")

    @functools.partial(
        pl.kernel,
        mesh=mesh,
        out_type=jax.ShapeDtypeStruct((_BATCH, _N_TOK, _D), jnp.float32),
        scratch_types=[
            pltpu.VMEM((b_per_w * _PAD_TOK,), jnp.int32),
            pltpu.VMEM((_PAD_TOK,), jnp.int32),
            pltpu.VMEM((_PAD_TOK, _D), jnp.float32),
            pltpu.VMEM((_PAD_TOK, _D), jnp.float32),
            pltpu.SemaphoreType.DMA,
            pltpu.SemaphoreType.DMA,
            pltpu.SemaphoreType.DMA,
            pltpu.SemaphoreType.DMA,
        ],
    )
    def body(
        tok_hbm, tab_hbm, pos_hbm, map_hbm, out_hbm,
        idx_v, map_v, buf, pos_v, g0, g1, g2, ws,
    ):
        wid = lax.axis_index("s") * info.num_cores + lax.axis_index("c")
        b0 = wid * b_per_w
        pltpu.sync_copy(tok_hbm.at[pl.ds(b0 * _PAD_TOK, b_per_w * _PAD_TOK)], idx_v)
        pltpu.sync_copy(map_hbm, map_v)
        pltpu.sync_copy(pos_hbm, pos_v)

        def start_gathers(j):
            o = j * _PAD_TOK
            pltpu.async_copy(
                tab_hbm.at[idx_v.at[pl.ds(o, 32)]], buf.at[pl.ds(0, 32)], g0
            )
            pltpu.async_copy(
                tab_hbm.at[idx_v.at[pl.ds(o + 32, 32)]], buf.at[pl.ds(32, 32)], g1
            )
            pltpu.async_copy(
                tab_hbm.at[idx_v.at[pl.ds(o + 64, 16)]], buf.at[pl.ds(64, 16)], g2
            )

        def wait_gather(rows, off, sem):
            pltpu.make_async_copy(
                tab_hbm.at[idx_v.at[pl.ds(0, rows)]],
                buf.at[pl.ds(off, rows)],
                sem,
            ).wait()

        def add_region(lo, hi):
            @plsc.parallel_loop(lo, hi)
            def _(i):
                @plsc.parallel_loop(0, _D // _LANES, unroll=8)
                def _(k):
                    sl = pl.ds(k * _LANES, _LANES)
                    buf[i, sl] = buf[i, sl] + pos_v[i, sl]

        def batch(j, carry):
            b = b0 + j
            start_gathers(j)
            wait_gather(32, 0, g0)
            add_region(0, 32)
            wait_gather(32, 32, g1)
            add_region(32, 64)
            wait_gather(16, 64, g2)
            add_region(64, _PAD_TOK)
            pltpu.async_copy(buf, out_hbm.at[b].at[map_v], ws)
            pltpu.make_async_copy(buf, out_hbm.at[b].at[map_v], ws).wait()
            return carry

        lax.fori_loop(0, b_per_w, batch, 0)

    return body(tokens_pflat, table, pos_pad, rowmap)


def kernel(tokens, token_embedding, positional_embedding):
    tokens_pad = jnp.concatenate(
        [tokens.astype(jnp.int32), tokens[:, 74:77].astype(jnp.int32)], axis=1
    ).reshape(-1)
    pos_pad = jnp.concatenate(
        [positional_embedding, positional_embedding[74:77]], axis=0
    )
    rowmap = jnp.concatenate(
        [jnp.arange(_N_TOK, dtype=jnp.int32),
         jnp.array([74, 75, 76], dtype=jnp.int32)]
    )
    return _sc_embed(tokens_pad, token_embedding, pos_pad, rowmap)


# speedup vs baseline: 1.1900x; 1.1096x over previous
import functools

import jax
import jax.numpy as jnp
from jax import lax
from jax.experimental import pallas as pl
from jax.experimental.pallas import tpu as pltpu
from jax.experimental.pallas import tpu_sc as plsc

_BATCH = 256
_N_TOK = 77
_PAD_TOK = 80
_D = 768
_LANES = 16
_REGIONS = ((0, 32), (32, 32), (64, 16))


def _sc_embed(tokens_pflat, table, pos_pad, rowmap):
    info = plsc.get_sparse_core_info()
    num_workers = info.num_cores * info.num_subcores
    b_per_w = _BATCH // num_workers

    mesh = plsc.VectorSubcoreMesh(core_axis_name="c", subcore_axis_name="s")

    @functools.partial(
        pl.kernel,
        mesh=mesh,
        out_type=jax.ShapeDtypeStruct((_BATCH, _N_TOK, _D), jnp.float32),
        scratch_types=[
            pltpu.VMEM((b_per_w * _PAD_TOK,), jnp.int32),
            pltpu.VMEM((32,), jnp.int32),
            pltpu.VMEM((32,), jnp.int32),
            pltpu.VMEM((16,), jnp.int32),
            pltpu.VMEM((_PAD_TOK, _D), jnp.float32),
            pltpu.VMEM((_PAD_TOK, _D), jnp.float32),
            pltpu.SemaphoreType.DMA,
            pltpu.SemaphoreType.DMA,
            pltpu.SemaphoreType.DMA,
            pltpu.SemaphoreType.DMA,
            pltpu.SemaphoreType.DMA,
            pltpu.SemaphoreType.DMA,
        ],
    )
    def body(
        tok_hbm, tab_hbm, pos_hbm, map_hbm, out_hbm,
        idx_v, mA, mB, mC, buf, pos_v,
        gA, gB, gC, sA, sB, sC,
    ):
        wid = lax.axis_index("s") * info.num_cores + lax.axis_index("c")
        b0 = wid * b_per_w
        pltpu.sync_copy(tok_hbm.at[pl.ds(b0 * _PAD_TOK, b_per_w * _PAD_TOK)], idx_v)
        pltpu.sync_copy(map_hbm.at[pl.ds(0, 32)], mA)
        pltpu.sync_copy(map_hbm.at[pl.ds(32, 32)], mB)
        pltpu.sync_copy(map_hbm.at[pl.ds(64, 16)], mC)
        pltpu.sync_copy(pos_hbm, pos_v)

        maps = (mA, mB, mC)
        gsems = (gA, gB, gC)
        ssems = (sA, sB, sC)

        def start_gather(j, r):
            o, n = _REGIONS[r]
            pltpu.async_copy(
                tab_hbm.at[idx_v.at[pl.ds(j * _PAD_TOK + o, n)]],
                buf.at[pl.ds(o, n)],
                gsems[r],
            )

        def wait_gather(r):
            o, n = _REGIONS[r]
            pltpu.make_async_copy(
                tab_hbm.at[idx_v.at[pl.ds(0, n)]], buf.at[pl.ds(o, n)], gsems[r]
            ).wait()

        def start_scatter(b, r):
            o, n = _REGIONS[r]
            pltpu.async_copy(
                buf.at[pl.ds(o, n)], out_hbm.at[b].at[maps[r]], ssems[r]
            )

        def wait_scatter(b, r):
            o, n = _REGIONS[r]
            pltpu.make_async_copy(
                buf.at[pl.ds(o, n)], out_hbm.at[b].at[maps[r]], ssems[r]
            ).wait()

        def add_region(r):
            o, n = _REGIONS[r]

            @plsc.parallel_loop(o, o + n)
            def _(i):
                @plsc.parallel_loop(0, _D // _LANES, unroll=8)
                def _(k):
                    sl = pl.ds(k * _LANES, _LANES)
                    buf[i, sl] = buf[i, sl] + pos_v[i, sl]

        def batch(j, carry):
            b = b0 + j
            for r in range(3):
                @pl.when(j > 0)
                def _():
                    wait_scatter(b - 1, r)

                start_gather(j, r)
            for r in range(3):
                wait_gather(r)
                add_region(r)
                start_scatter(b, r)
            return carry

        lax.fori_loop(0, b_per_w, batch, 0)
        for r in range(3):
            wait_scatter(b0 + b_per_w - 1, r)

    return body(tokens_pflat, table, pos_pad, rowmap)


def kernel(tokens, token_embedding, positional_embedding):
    tokens_pad = jnp.concatenate(
        [tokens.astype(jnp.int32), tokens[:, 74:77].astype(jnp.int32)], axis=1
    ).reshape(-1)
    pos_pad = jnp.concatenate(
        [positional_embedding, positional_embedding[74:77]], axis=0
    )
    rowmap = jnp.concatenate(
        [jnp.arange(_N_TOK, dtype=jnp.int32),
         jnp.array([74, 75, 76], dtype=jnp.int32)]
    )
    return _sc_embed(tokens_pad, token_embedding, pos_pad, rowmap)


# speedup vs baseline: 1.1915x; 1.0012x over previous
import functools

import jax
import jax.numpy as jnp
from jax import lax
from jax.experimental import pallas as pl
from jax.experimental.pallas import tpu as pltpu
from jax.experimental.pallas import tpu_sc as plsc

_BATCH = 256
_N_TOK = 77
_PAD_TOK = 80
_D = 768
_LANES = 16
_REGIONS = ((0, 32), (32, 32), (64, 16))


def _sc_embed(tokens_pflat, table, pos, pos_tail):
    info = plsc.get_sparse_core_info()
    num_workers = info.num_cores * info.num_subcores
    b_per_w = _BATCH // num_workers

    mesh = plsc.VectorSubcoreMesh(core_axis_name="c", subcore_axis_name="s")

    @functools.partial(
        pl.kernel,
        mesh=mesh,
        out_type=jax.ShapeDtypeStruct((_BATCH, _N_TOK, _D), jnp.float32),
        scratch_types=[
            pltpu.VMEM((b_per_w * _PAD_TOK,), jnp.int32),
            pltpu.VMEM((32,), jnp.int32),
            pltpu.VMEM((32,), jnp.int32),
            pltpu.VMEM((16,), jnp.int32),
            pltpu.VMEM((_PAD_TOK, _D), jnp.float32),
            pltpu.VMEM((_PAD_TOK, _D), jnp.float32),
            pltpu.SemaphoreType.DMA,
            pltpu.SemaphoreType.DMA,
            pltpu.SemaphoreType.DMA,
            pltpu.SemaphoreType.DMA,
            pltpu.SemaphoreType.DMA,
            pltpu.SemaphoreType.DMA,
        ],
    )
    def body(
        tok_hbm, tab_hbm, pos_hbm, ptail_hbm, out_hbm,
        idx_v, mA, mB, mC, buf, pos_v,
        gA, gB, gC, sA, sB, sC,
    ):
        wid = lax.axis_index("s") * info.num_cores + lax.axis_index("c")
        b0 = wid * b_per_w

        lanes = lax.iota(jnp.int32, _LANES)
        fold = lanes - jnp.where(lanes >= 13, 3, 0)
        mA[pl.ds(0, _LANES)] = lanes
        mA[pl.ds(_LANES, _LANES)] = lanes + _LANES
        mB[pl.ds(0, _LANES)] = lanes + 32
        mB[pl.ds(_LANES, _LANES)] = lanes + 48
        mC[pl.ds(0, _LANES)] = fold + 64

        pltpu.sync_copy(pos_hbm.at[pl.ds(0, 72)], pos_v.at[pl.ds(0, 72)])
        pltpu.sync_copy(ptail_hbm, pos_v.at[pl.ds(72, 8)])

        pltpu.sync_copy(
            tok_hbm.at[pl.ds(b0 * _PAD_TOK, b_per_w * _PAD_TOK)], idx_v
        )

        maps = (mA, mB, mC)
        gsems = (gA, gB, gC)
        ssems = (sA, sB, sC)

        def start_gather(j, r):
            o, n = _REGIONS[r]
            pltpu.async_copy(
                tab_hbm.at[idx_v.at[pl.ds(j * _PAD_TOK + o, n)]],
                buf.at[pl.ds(o, n)],
                gsems[r],
            )

        def wait_gather(r):
            o, n = _REGIONS[r]
            pltpu.make_async_copy(
                tab_hbm.at[idx_v.at[pl.ds(0, n)]], buf.at[pl.ds(o, n)], gsems[r]
            ).wait()

        def start_scatter(b, r):
            o, n = _REGIONS[r]
            pltpu.async_copy(
                buf.at[pl.ds(o, n)], out_hbm.at[b].at[maps[r]], ssems[r]
            )

        def wait_scatter(b, r):
            o, n = _REGIONS[r]
            pltpu.make_async_copy(
                buf.at[pl.ds(o, n)], out_hbm.at[b].at[maps[r]], ssems[r]
            ).wait()

        def add_region(r):
            o, n = _REGIONS[r]

            @plsc.parallel_loop(o, o + n)
            def _(i):
                @plsc.parallel_loop(0, _D // _LANES, unroll=8)
                def _(k):
                    sl = pl.ds(k * _LANES, _LANES)
                    buf[i, sl] = buf[i, sl] + pos_v[i, sl]

        def batch(j, carry):
            b = b0 + j
            for r in range(3):
                @pl.when(j > 0)
                def _():
                    wait_scatter(b - 1, r)

                start_gather(j, r)
            for r in range(3):
                wait_gather(r)
                add_region(r)
                start_scatter(b, r)
            return carry

        lax.fori_loop(0, b_per_w, batch, 0)
        for r in range(3):
            wait_scatter(b0 + b_per_w - 1, r)

    return body(tokens_pflat, table, pos, pos_tail)


def kernel(tokens, token_embedding, positional_embedding):
    tokens_pad = jnp.concatenate(
        [tokens.astype(jnp.int32), tokens[:, 74:77].astype(jnp.int32)], axis=1
    ).reshape(-1)
    tail_rows = jnp.array([72, 73, 74, 75, 76, 74, 75, 76], dtype=jnp.int32)
    pos_tail = jnp.take(positional_embedding, tail_rows, axis=0)
    return _sc_embed(
        tokens_pad, token_embedding, positional_embedding, pos_tail
    )


# speedup vs baseline: 1.2448x; 1.0447x over previous
import functools

import jax
import jax.numpy as jnp
from jax import lax
from jax.experimental import pallas as pl
from jax.experimental.pallas import tpu as pltpu
from jax.experimental.pallas import tpu_sc as plsc

_BATCH = 256
_N_TOK = 77
_PAD_TOK = 80
_D = 768
_LANES = 16
_REGIONS = ((0, 32), (32, 48))


def _sc_embed(tokens_pflat, table, pos, pos_tail):
    info = plsc.get_sparse_core_info()
    num_workers = info.num_cores * info.num_subcores
    b_per_w = _BATCH // num_workers

    mesh = plsc.VectorSubcoreMesh(core_axis_name="c", subcore_axis_name="s")

    @functools.partial(
        pl.kernel,
        mesh=mesh,
        out_type=jax.ShapeDtypeStruct((_BATCH, _N_TOK, _D), jnp.float32),
        scratch_types=[
            pltpu.VMEM((b_per_w * _PAD_TOK,), jnp.int32),
            pltpu.VMEM((32,), jnp.int32),
            pltpu.VMEM((48,), jnp.int32),
            pltpu.VMEM((_PAD_TOK, _D), jnp.float32),
            pltpu.VMEM((_PAD_TOK, _D), jnp.float32),
            pltpu.SemaphoreType.DMA,
            pltpu.SemaphoreType.DMA,
            pltpu.SemaphoreType.DMA,
            pltpu.SemaphoreType.DMA,
        ],
    )
    def body(
        tok_hbm, tab_hbm, pos_hbm, ptail_hbm, out_hbm,
        idx_v, mA, mB, buf, pos_v,
        gA, gB, sA, sB,
    ):
        wid = lax.axis_index("s") * info.num_cores + lax.axis_index("c")
        b0 = wid * b_per_w

        lanes = lax.iota(jnp.int32, _LANES)
        fold = lanes - jnp.where(lanes >= 13, 3, 0)
        mA[pl.ds(0, _LANES)] = lanes
        mA[pl.ds(_LANES, _LANES)] = lanes + _LANES
        mB[pl.ds(0, _LANES)] = lanes + 32
        mB[pl.ds(_LANES, _LANES)] = lanes + 48
        mB[pl.ds(2 * _LANES, _LANES)] = fold + 64

        pltpu.sync_copy(pos_hbm.at[pl.ds(0, 72)], pos_v.at[pl.ds(0, 72)])
        pltpu.sync_copy(ptail_hbm, pos_v.at[pl.ds(72, 8)])

        pltpu.sync_copy(
            tok_hbm.at[pl.ds(b0 * _PAD_TOK, b_per_w * _PAD_TOK)], idx_v
        )

        maps = (mA, mB)
        gsems = (gA, gB)
        ssems = (sA, sB)

        def start_gather(j, r):
            o, n = _REGIONS[r]
            pltpu.async_copy(
                tab_hbm.at[idx_v.at[pl.ds(j * _PAD_TOK + o, n)]],
                buf.at[pl.ds(o, n)],
                gsems[r],
            )

        def wait_gather(r):
            o, n = _REGIONS[r]
            pltpu.make_async_copy(
                tab_hbm.at[idx_v.at[pl.ds(0, n)]], buf.at[pl.ds(o, n)], gsems[r]
            ).wait()

        def start_scatter(b, r):
            o, n = _REGIONS[r]
            pltpu.async_copy(
                buf.at[pl.ds(o, n)], out_hbm.at[b].at[maps[r]], ssems[r]
            )

        def wait_scatter(b, r):
            o, n = _REGIONS[r]
            pltpu.make_async_copy(
                buf.at[pl.ds(o, n)], out_hbm.at[b].at[maps[r]], ssems[r]
            ).wait()

        def add_region(r):
            o, n = _REGIONS[r]

            @plsc.parallel_loop(o, o + n)
            def _(i):
                @plsc.parallel_loop(0, _D // _LANES, unroll=8)
                def _(k):
                    sl = pl.ds(k * _LANES, _LANES)
                    buf[i, sl] = buf[i, sl] + pos_v[i, sl]

        def batch(j, carry):
            b = b0 + j
            for r in range(len(_REGIONS)):
                @pl.when(j > 0)
                def _():
                    wait_scatter(b - 1, r)

                start_gather(j, r)
            for r in range(len(_REGIONS)):
                wait_gather(r)
                add_region(r)
                start_scatter(b, r)
            return carry

        lax.fori_loop(0, b_per_w, batch, 0)
        for r in range(len(_REGIONS)):
            wait_scatter(b0 + b_per_w - 1, r)

    return body(tokens_pflat, table, pos, pos_tail)


def kernel(tokens, token_embedding, positional_embedding):
    tokens_pad = jnp.concatenate(
        [tokens.astype(jnp.int32), tokens[:, 74:77].astype(jnp.int32)], axis=1
    ).reshape(-1)
    tail_rows = jnp.array([72, 73, 74, 75, 76, 74, 75, 76], dtype=jnp.int32)
    pos_tail = jnp.take(positional_embedding, tail_rows, axis=0)
    return _sc_embed(
        tokens_pad, token_embedding, positional_embedding, pos_tail
    )
